# baseline (device time: 50606 ns/iter reference)
import jax
import jax.numpy as jnp
from jax import lax
from jax.experimental import pallas as pl
from jax.experimental.pallas import tpu as pltpu

N_DEV = 4


def kernel(A, B):
    m, k = A.shape
    k2, n = B.shape
    assert k == k2
    m_per = m // N_DEV
    h_per = m_per // 2

    f32 = jnp.float32
    bf16 = jnp.bfloat16

    def body(a_ref, b_ref, out_ref, part_ref, b16_ref,
             x1f_snd, x1f_rcv, x1o_snd, x1o_rcv,
             y1f_snd, y1f_rcv, y1o_snd, y1o_rcv,
             x2_snd, x2_rcv, y2_snd, y2_rcv,
             send_sems, recv_sems):
        my = lax.axis_index("i")
        xp = 3 - my
        yp = my ^ 1
        xyp = 3 - (my ^ 1)
        yxp = (3 - my) ^ 1

        barrier_sem = pltpu.get_barrier_semaphore()
        for nbr in [xp, yp]:
            pl.semaphore_signal(
                barrier_sem, inc=1,
                device_id=(nbr,), device_id_type=pl.DeviceIdType.MESH,
            )
        pl.semaphore_wait(barrier_sem, 2)

        def matmul(start, nrows):
            return jnp.dot(
                a_ref[pl.ds(start, nrows), :], b_ref[:, :],
                preferred_element_type=f32,
            )

        def matmul_bf16(start, nrows):
            return jnp.dot(
                a_ref[pl.ds(start, nrows), :].astype(bf16), b16_ref[:, :],
                preferred_element_type=f32,
            )

        def lrow(c):
            return c * m_per

        def rrow(c):
            return c * m_per + h_per

        PYR = 0
        PXL = h_per
        MYL = 2 * h_per
        MYR = 3 * h_per

        def rdma(src, dst, i, tgt):
            return pltpu.make_async_remote_copy(
                src_ref=src, dst_ref=dst,
                send_sem=send_sems.at[i], recv_sem=recv_sems.at[i],
                device_id=(tgt,), device_id_type=pl.DeviceIdType.MESH,
            )

        b16_ref[:, :] = b_ref[:, :].astype(bf16)
        x1f_snd[:, :] = matmul_bf16(rrow(yxp), h_per).astype(bf16)
        r_x1f = rdma(x1f_snd, x1f_rcv, 0, xp)
        r_x1f.start()
        y1f_snd[:, :] = matmul_bf16(lrow(xyp), h_per).astype(bf16)
        r_y1f = rdma(y1f_snd, y1f_rcv, 1, yp)
        r_y1f.start()
        x1o_snd[:, :] = matmul_bf16(rrow(xp), h_per).astype(bf16)
        r_x1o = rdma(x1o_snd, x1o_rcv, 2, xp)
        r_x1o.start()
        y1o_snd[:, :] = matmul_bf16(lrow(yp), h_per).astype(bf16)
        r_y1o = rdma(y1o_snd, y1o_rcv, 3, yp)
        r_y1o.start()

        part_ref[pl.ds(PYR, h_per), :] = matmul(rrow(yp), h_per)
        part_ref[pl.ds(PXL, h_per), :] = matmul(lrow(xp), h_per)

        r_x1f.wait()
        y2_snd[:, :] = (
            part_ref[pl.ds(PYR, h_per), :] + x1f_rcv[:, :].astype(f32)
        ).astype(bf16)
        r_y2 = rdma(y2_snd, y2_rcv, 4, yp)
        r_y2.start()

        r_y1f.wait()
        x2_snd[:, :] = (
            part_ref[pl.ds(PXL, h_per), :] + y1f_rcv[:, :].astype(f32)
        ).astype(bf16)
        r_x2 = rdma(x2_snd, x2_rcv, 5, xp)
        r_x2.start()

        part_ref[pl.ds(MYL, m_per), :] = matmul(lrow(my), m_per)

        r_x1o.wait()
        part_ref[pl.ds(MYR, h_per), :] = (
            part_ref[pl.ds(MYR, h_per), :] + x1o_rcv[:, :].astype(f32)
        )
        r_y1o.wait()
        part_ref[pl.ds(MYL, h_per), :] = (
            part_ref[pl.ds(MYL, h_per), :] + y1o_rcv[:, :].astype(f32)
        )

        r_x2.wait()
        out_ref[0:h_per, :] = (
            part_ref[pl.ds(MYL, h_per), :] + x2_rcv[:, :].astype(f32)
        )
        r_y2.wait()
        out_ref[h_per:m_per, :] = (
            part_ref[pl.ds(MYR, h_per), :] + y2_rcv[:, :].astype(f32)
        )

    comm = [pltpu.VMEM((h_per, n), bf16) for _ in range(12)]
    return pl.pallas_call(
        body,
        out_shape=jax.ShapeDtypeStruct((m_per, n), f32),
        in_specs=[
            pl.BlockSpec(memory_space=pltpu.VMEM),
            pl.BlockSpec(memory_space=pltpu.VMEM),
        ],
        out_specs=pl.BlockSpec(memory_space=pltpu.VMEM),
        scratch_shapes=[
            pltpu.VMEM((4 * h_per, n), f32),
            pltpu.VMEM((k, n), bf16),
            *comm,
            pltpu.SemaphoreType.DMA((6,)),
            pltpu.SemaphoreType.DMA((6,)),
        ],
        compiler_params=pltpu.CompilerParams(collective_id=0),
    )(A, B)


# device time: 50577 ns/iter; 1.0006x vs baseline; 1.0006x over previous
import jax
import jax.numpy as jnp
from jax import lax
from jax.experimental import pallas as pl
from jax.experimental.pallas import tpu as pltpu

N_DEV = 4


def kernel(A, B):
    m, k = A.shape
    k2, n = B.shape
    assert k == k2
    m_per = m // N_DEV
    h_per = m_per // 2

    f32 = jnp.float32
    bf16 = jnp.bfloat16

    def body(a_ref, b_ref, out_ref, part_ref,
             x1f_snd, x1f_rcv, x1o_snd, x1o_rcv,
             y1f_snd, y1f_rcv, y1o_snd, y1o_rcv,
             x2_snd, x2_rcv, y2_snd, y2_rcv,
             send_sems, recv_sems):
        my = lax.axis_index("i")
        xp = 3 - my
        yp = my ^ 1
        xyp = 3 - (my ^ 1)
        yxp = (3 - my) ^ 1

        barrier_sem = pltpu.get_barrier_semaphore()
        for nbr in [xp, yp]:
            pl.semaphore_signal(
                barrier_sem, inc=1,
                device_id=(nbr,), device_id_type=pl.DeviceIdType.MESH,
            )
        pl.semaphore_wait(barrier_sem, 2)

        def matmul(start, nrows):
            return jnp.dot(
                a_ref[pl.ds(start, nrows), :], b_ref[:, :],
                preferred_element_type=f32,
            )

        def lrow(c):
            return c * m_per

        def rrow(c):
            return c * m_per + h_per

        def rdma(src, dst, i, tgt):
            return pltpu.make_async_remote_copy(
                src_ref=src, dst_ref=dst,
                send_sem=send_sems.at[i], recv_sem=recv_sems.at[i],
                device_id=(tgt,), device_id_type=pl.DeviceIdType.MESH,
            )

        x1f_snd[:, :] = matmul(rrow(yxp), h_per).astype(bf16)
        r_x1f = rdma(x1f_snd, x1f_rcv, 0, xp)
        r_x1f.start()
        y1f_snd[:, :] = matmul(lrow(xyp), h_per).astype(bf16)
        r_y1f = rdma(y1f_snd, y1f_rcv, 1, yp)
        r_y1f.start()
        x1o_snd[:, :] = matmul(rrow(xp), h_per).astype(bf16)
        r_x1o = rdma(x1o_snd, x1o_rcv, 2, xp)
        r_x1o.start()
        y1o_snd[:, :] = matmul(lrow(yp), h_per).astype(bf16)
        r_y1o = rdma(y1o_snd, y1o_rcv, 3, yp)
        r_y1o.start()

        part_ref[pl.ds(rrow(yp), h_per), :] = matmul(rrow(yp), h_per)
        part_ref[pl.ds(lrow(xp), h_per), :] = matmul(lrow(xp), h_per)

        r_x1f.wait()
        y2_snd[:, :] = (
            part_ref[pl.ds(rrow(yp), h_per), :] + x1f_rcv[:, :].astype(f32)
        ).astype(bf16)
        r_y2 = rdma(y2_snd, y2_rcv, 4, yp)
        r_y2.start()

        r_y1f.wait()
        x2_snd[:, :] = (
            part_ref[pl.ds(lrow(xp), h_per), :] + y1f_rcv[:, :].astype(f32)
        ).astype(bf16)
        r_x2 = rdma(x2_snd, x2_rcv, 5, xp)
        r_x2.start()

        part_ref[pl.ds(lrow(my), m_per), :] = matmul(lrow(my), m_per)

        r_x1o.wait()
        part_ref[pl.ds(rrow(my), h_per), :] = (
            part_ref[pl.ds(rrow(my), h_per), :] + x1o_rcv[:, :].astype(f32)
        )
        r_y1o.wait()
        part_ref[pl.ds(lrow(my), h_per), :] = (
            part_ref[pl.ds(lrow(my), h_per), :] + y1o_rcv[:, :].astype(f32)
        )

        r_x2.wait()
        out_ref[0:h_per, :] = (
            part_ref[pl.ds(lrow(my), h_per), :] + x2_rcv[:, :].astype(f32)
        )
        r_y2.wait()
        out_ref[h_per:m_per, :] = (
            part_ref[pl.ds(rrow(my), h_per), :] + y2_rcv[:, :].astype(f32)
        )

    comm = [pltpu.VMEM((h_per, n), bf16) for _ in range(12)]
    return pl.pallas_call(
        body,
        out_shape=jax.ShapeDtypeStruct((m_per, n), f32),
        in_specs=[
            pl.BlockSpec(memory_space=pltpu.VMEM),
            pl.BlockSpec(memory_space=pltpu.VMEM),
        ],
        out_specs=pl.BlockSpec(memory_space=pltpu.VMEM),
        scratch_shapes=[
            pltpu.VMEM((m, n), f32),
            *comm,
            pltpu.SemaphoreType.DMA((6,)),
            pltpu.SemaphoreType.DMA((6,)),
        ],
        compiler_params=pltpu.CompilerParams(collective_id=0),
    )(A, B)
